# baseline (speedup 1.0000x reference)
import functools

import jax
import jax.numpy as jnp
from jax import lax
from jax.experimental import pallas as pl
from jax.experimental.pallas import tpu as pltpu
from jax.experimental.pallas import tpu_sc as plsc

N = 10000
E = 320000
D = 128
NCT = 20

NC = 2
NS = 16
NW = NC * NS
L = 16

CHUNK = 120
NBUF = 3
G = 4
NSLAB = 3
SUPER = G * NSLAB
CPT0 = 96
CPT1 = 72
NGRP_MAX = max(CPT0, CPT1) // G
EP = (CPT0 + CPT1) * CHUNK * NS
DEG_EPT = E // NW

N_PAD = 10112
RPT = N_PAD // NS

_mesh = plsc.VectorSubcoreMesh(core_axis_name="c", subcore_axis_name="s")
_sc_params = pltpu.CompilerParams(needs_layout_passes=False)


@functools.partial(
    pl.kernel,
    out_type=jax.ShapeDtypeStruct((NW, N_PAD), jnp.float32),
    mesh=_mesh,
    compiler_params=_sc_params,
    scratch_types=[
        pltpu.VMEM((DEG_EPT,), jnp.int32),
        pltpu.VMEM((N_PAD,), jnp.float32),
    ],
)
def _deg_kernel(dst_hbm, out_hbm, dst_v, hist_v):
    wid = lax.axis_index("s") * NC + lax.axis_index("c")
    zeros = jnp.zeros((L,), jnp.float32)

    def zero_body(i, _):
        hist_v[pl.ds(i * L, L)] = zeros
        return 0

    lax.fori_loop(0, N_PAD // L, zero_body, 0)
    pltpu.sync_copy(dst_hbm.at[pl.ds(wid * DEG_EPT, DEG_EPT)], dst_v)
    ones = jnp.ones((L,), jnp.float32)

    def body(i, _):
        idx = dst_v[pl.ds(i * L, L)]
        plsc.addupdate_scatter(hist_v, [idx], ones)
        return 0

    lax.fori_loop(0, DEG_EPT // L, body, 0)
    pltpu.sync_copy(hist_v, out_hbm.at[wid])


@functools.partial(
    pl.kernel,
    out_type=jax.ShapeDtypeStruct((NC, N_PAD, D), jnp.float32),
    mesh=_mesh,
    compiler_params=_sc_params,
    scratch_types=[pltpu.VMEM_SHARED((N_PAD, D), jnp.float32)]
    + [pltpu.VMEM((G, 2, CHUNK), jnp.int32)] * NSLAB
    + [pltpu.VMEM((CHUNK, D), jnp.float32)] * NBUF
    + [pltpu.SemaphoreType.DMA] * (2 * NBUF + NSLAB),
)
def _prop_kernel(t_hbm, idx_hbm, zeros_hbm, out_hbm, acc_sh, *scratch):
    islabs = scratch[:NSLAB]
    rows = scratch[NSLAB:NSLAB + NBUF]
    gsems = scratch[NSLAB + NBUF:NSLAB + 2 * NBUF]
    ssems = scratch[NSLAB + 2 * NBUF:NSLAB + 3 * NBUF]
    isems = scratch[NSLAB + 3 * NBUF:]
    c = lax.axis_index("c")
    s = lax.axis_index("s")
    wid = s * NC + c

    def islab_fetch(g, p):
        pltpu.async_copy(idx_hbm.at[wid, g], islabs[p], isems[p])

    def islab_wait(g, p):
        pltpu.make_async_copy(idx_hbm.at[wid, g], islabs[p], isems[p]).wait()

    def g_start(j, p, b):
        pltpu.async_copy(t_hbm.at[islabs[p].at[j, 0]], rows[b], gsems[b])

    def g_wait(j, p, b):
        pltpu.make_async_copy(t_hbm.at[islabs[p].at[j, 0]], rows[b],
                              gsems[b]).wait()

    def s_start(j, p, b):
        pltpu.async_copy(rows[b], acc_sh.at[islabs[p].at[j, 1]], ssems[b],
                         add=True)

    def s_wait(j, p, b):
        pltpu.make_async_copy(rows[b], acc_sh.at[islabs[p].at[j, 1]],
                              ssems[b]).wait()

    pltpu.sync_copy(zeros_hbm, acc_sh.at[pl.ds(s * RPT, RPT)])
    plsc.subcore_barrier()
    islab_fetch(0, 0)
    islab_fetch(1, 1)
    islab_wait(0, 0)
    g_start(0, 0, 0)

    def emit_chunk(i, jj, last_super, first_super=False):
        p = (jj // G) % NSLAB
        j = jj % G
        b = jj % NBUF
        g_wait(j, p, b)
        s_start(j, p, b)
        if not (first_super and jj < 2):
            jp = (jj - 2) % SUPER
            s_wait(jp % G, (jp // G) % NSLAB, jp % NBUF)
        if jj % G == 1 and not (last_super and jj > 1):
            islab_fetch(i * NSLAB + jj // G + 2, (jj // G + 2) % NSLAB)
        if not (last_super and jj == SUPER - 1):
            jn = (jj + 1) % SUPER
            pn = (jn // G) % NSLAB
            if (jj + 1) % G == 0:
                islab_wait(i * NSLAB + (jj + 1) // G, pn)
            g_start(jn % G, pn, (jj + 1) % NBUF)

    def body(i, _):
        for jj in range(SUPER):
            emit_chunk(i, jj, False)
        return 0

    nsup = jnp.where(c == 0, CPT0 // SUPER, CPT1 // SUPER)
    for jj in range(SUPER):
        emit_chunk(0, jj, False, first_super=True)
    lax.fori_loop(1, nsup - 1, body, 0)
    for jj in range(SUPER):
        emit_chunk(nsup - 1, jj, True)
    for jp in (SUPER - 2, SUPER - 1):
        s_wait(jp % G, (jp // G) % NSLAB, jp % NBUF)

    plsc.subcore_barrier()
    pltpu.sync_copy(acc_sh.at[pl.ds(s * RPT, RPT)],
                    out_hbm.at[c, pl.ds(s * RPT, RPT)])


def _prep_body(x_ref, win_ref, w1_ref, parts_ref, t1_ref, dis_ref):
    h0 = jnp.dot(x_ref[...], win_ref[...], preferred_element_type=jnp.float32)
    hw1 = jnp.dot(h0, w1_ref[...], preferred_element_type=jnp.float32)
    deg = jnp.sum(parts_ref[...], axis=0) + 1.0
    dis = lax.rsqrt(deg)[:, None]
    dis_ref[...] = dis
    t1_ref[...] = dis * hw1


def _mid_body(p_ref, t_ref, dis_ref, b_ref, w_ref, tn_ref):
    dis = dis_ref[...]
    h = dis * (p_ref[0] + p_ref[1] + t_ref[...]) + b_ref[...]
    h = jnp.maximum(h, 0.0)
    tn_ref[...] = dis * jnp.dot(h, w_ref[...],
                                preferred_element_type=jnp.float32)


def _final_body(q_ref, t_ref, dis_ref, b_ref, wm1_ref, bm1_ref, gamma_ref,
                beta_ref, wm2_ref, bm2_ref, out_ref):
    dis = dis_ref[...]
    h = dis * (q_ref[0] + q_ref[1] + t_ref[...]) + b_ref[...]
    h = jnp.maximum(h, 0.0)[:N]
    m = jnp.dot(h, wm1_ref[...], preferred_element_type=jnp.float32)
    m = m + bm1_ref[...]
    mu = jnp.mean(m, axis=0)
    var = jnp.mean((m - mu) ** 2, axis=0)
    m = gamma_ref[...] * (m - mu) * lax.rsqrt(var + 1e-5) + beta_ref[...]
    m = jnp.maximum(m, 0.0)
    logits = jnp.dot(m, wm2_ref[...], preferred_element_type=jnp.float32)
    logits = logits + bm2_ref[...]
    logits = logits - jnp.max(logits, axis=-1, keepdims=True)
    ex = jnp.exp(logits)
    out_ref[...] = ex / jnp.sum(ex, axis=-1, keepdims=True)


_vmem = pl.BlockSpec(memory_space=pltpu.VMEM)


def _tc_call(body, num_in, out_shape):
    return pl.pallas_call(
        body,
        in_specs=[_vmem] * num_in,
        out_specs=_vmem if not isinstance(out_shape, (tuple, list)) else
        [_vmem] * len(out_shape),
        out_shape=out_shape,
    )


def kernel(x, edge_index, W_in, W1, b1, W2, b2, Wm1, bm1, gamma, beta,
           Wm2, bm2):
    src = edge_index[0].astype(jnp.int32)
    dst = edge_index[1].astype(jnp.int32)
    pad = jnp.full((EP - E,), N, dtype=jnp.int32)
    src_p = jnp.concatenate([src, pad])
    dst_p = jnp.concatenate([dst, pad])
    wid_slabs = []
    off = 0
    for _s in range(NS):
        for _c in range(NC):
            cpt = CPT0 if _c == 0 else CPT1
            seg = jnp.stack(
                [src_p[off:off + cpt * CHUNK].reshape(-1, G, CHUNK),
                 dst_p[off:off + cpt * CHUNK].reshape(-1, G, CHUNK)], axis=2)
            seg = jnp.pad(seg, ((0, NGRP_MAX - cpt // G), (0, 0), (0, 0),
                                (0, 0)), constant_values=N)
            wid_slabs.append(seg)
            off += cpt * CHUNK
    idx5 = jnp.stack(wid_slabs)
    x_p = jnp.pad(x, ((0, N_PAD - N), (0, 0)))
    zrows = jnp.zeros((RPT, D), jnp.float32)

    parts = _deg_kernel(dst)

    t1, dis = _tc_call(
        _prep_body, 4,
        [jax.ShapeDtypeStruct((N_PAD, D), jnp.float32),
         jax.ShapeDtypeStruct((N_PAD, 1), jnp.float32)],
    )(x_p, W_in, W1, parts)

    p1 = _prop_kernel(t1, idx5, zrows)

    t2 = _tc_call(
        _mid_body, 5, jax.ShapeDtypeStruct((N_PAD, D), jnp.float32),
    )(p1, t1, dis, b1, W2)

    p2 = _prop_kernel(t2, idx5, zrows)

    out = _tc_call(
        _final_body, 10, jax.ShapeDtypeStruct((N, NCT), jnp.float32),
    )(p2, t2, dis, b2, Wm1, bm1, gamma, beta, Wm2, bm2)
    return out


# speedup vs baseline: 1.1879x; 1.1879x over previous
import functools

import jax
import jax.numpy as jnp
from jax import lax
from jax.experimental import pallas as pl
from jax.experimental.pallas import tpu as pltpu
from jax.experimental.pallas import tpu_sc as plsc

N = 10000
E = 320000
D = 128
NCT = 20

NC = 2
NS = 16
NW = NC * NS
L = 16

CHUNK = 120
NBUF = 3
G = 4
NSLAB = 3
SUPER = G * NSLAB
CPT0 = 120
CPT1 = 48
NGRP_MAX = max(CPT0, CPT1) // G
EP = (CPT0 + CPT1) * CHUNK * NS
DEG_EPT = E // NW

N_PAD = 10112
RPT = N_PAD // NS

_mesh = plsc.VectorSubcoreMesh(core_axis_name="c", subcore_axis_name="s")
_sc_params = pltpu.CompilerParams(needs_layout_passes=False)


@functools.partial(
    pl.kernel,
    out_type=jax.ShapeDtypeStruct((NW, N_PAD), jnp.float32),
    mesh=_mesh,
    compiler_params=_sc_params,
    scratch_types=[
        pltpu.VMEM((DEG_EPT,), jnp.int32),
        pltpu.VMEM((N_PAD,), jnp.float32),
    ],
)
def _deg_kernel(dst_hbm, out_hbm, dst_v, hist_v):
    wid = lax.axis_index("s") * NC + lax.axis_index("c")
    zeros = jnp.zeros((L,), jnp.float32)

    def zero_body(i, _):
        hist_v[pl.ds(i * L, L)] = zeros
        return 0

    lax.fori_loop(0, N_PAD // L, zero_body, 0)
    pltpu.sync_copy(dst_hbm.at[pl.ds(wid * DEG_EPT, DEG_EPT)], dst_v)
    ones = jnp.ones((L,), jnp.float32)

    def body(i, _):
        idx = dst_v[pl.ds(i * L, L)]
        plsc.addupdate_scatter(hist_v, [idx], ones)
        return 0

    lax.fori_loop(0, DEG_EPT // L, body, 0)
    pltpu.sync_copy(hist_v, out_hbm.at[wid])


@functools.partial(
    pl.kernel,
    out_type=jax.ShapeDtypeStruct((NC, N_PAD, D), jnp.float32),
    mesh=_mesh,
    compiler_params=_sc_params,
    scratch_types=[pltpu.VMEM_SHARED((N_PAD, D), jnp.float32)]
    + [pltpu.VMEM((G, 2, CHUNK), jnp.int32)] * NSLAB
    + [pltpu.VMEM((CHUNK, D), jnp.float32)] * NBUF
    + [pltpu.SemaphoreType.DMA] * (2 * NBUF + NSLAB),
)
def _prop_kernel(t_hbm, idx_hbm, zeros_hbm, out_hbm, acc_sh, *scratch):
    islabs = scratch[:NSLAB]
    rows = scratch[NSLAB:NSLAB + NBUF]
    gsems = scratch[NSLAB + NBUF:NSLAB + 2 * NBUF]
    ssems = scratch[NSLAB + 2 * NBUF:NSLAB + 3 * NBUF]
    isems = scratch[NSLAB + 3 * NBUF:]
    c = lax.axis_index("c")
    s = lax.axis_index("s")
    wid = s * NC + c

    def islab_fetch(g, p):
        pltpu.async_copy(idx_hbm.at[wid, g], islabs[p], isems[p])

    def islab_wait(g, p):
        pltpu.make_async_copy(idx_hbm.at[wid, g], islabs[p], isems[p]).wait()

    def g_start(j, p, b):
        pltpu.async_copy(t_hbm.at[islabs[p].at[j, 0]], rows[b], gsems[b])

    def g_wait(j, p, b):
        pltpu.make_async_copy(t_hbm.at[islabs[p].at[j, 0]], rows[b],
                              gsems[b]).wait()

    def s_start(j, p, b):
        pltpu.async_copy(rows[b], acc_sh.at[islabs[p].at[j, 1]], ssems[b],
                         add=True)

    def s_wait(j, p, b):
        pltpu.make_async_copy(rows[b], acc_sh.at[islabs[p].at[j, 1]],
                              ssems[b]).wait()

    pltpu.sync_copy(zeros_hbm, acc_sh.at[pl.ds(s * RPT, RPT)])
    plsc.subcore_barrier()
    islab_fetch(0, 0)
    islab_fetch(1, 1)
    islab_wait(0, 0)
    g_start(0, 0, 0)

    def emit_chunk(i, jj, last_super, first_super=False):
        p = (jj // G) % NSLAB
        j = jj % G
        b = jj % NBUF
        g_wait(j, p, b)
        s_start(j, p, b)
        if not (first_super and jj < 2):
            jp = (jj - 2) % SUPER
            s_wait(jp % G, (jp // G) % NSLAB, jp % NBUF)
        if jj % G == 1 and not (last_super and jj > 1):
            islab_fetch(i * NSLAB + jj // G + 2, (jj // G + 2) % NSLAB)
        if not (last_super and jj == SUPER - 1):
            jn = (jj + 1) % SUPER
            pn = (jn // G) % NSLAB
            if (jj + 1) % G == 0:
                islab_wait(i * NSLAB + (jj + 1) // G, pn)
            g_start(jn % G, pn, (jj + 1) % NBUF)

    def body(i, _):
        for jj in range(SUPER):
            emit_chunk(i, jj, False)
        return 0

    nsup = jnp.where(c == 0, CPT0 // SUPER, CPT1 // SUPER)
    for jj in range(SUPER):
        emit_chunk(0, jj, False, first_super=True)
    lax.fori_loop(1, nsup - 1, body, 0)
    for jj in range(SUPER):
        emit_chunk(nsup - 1, jj, True)
    for jp in (SUPER - 2, SUPER - 1):
        s_wait(jp % G, (jp // G) % NSLAB, jp % NBUF)

    plsc.subcore_barrier()
    pltpu.sync_copy(acc_sh.at[pl.ds(s * RPT, RPT)],
                    out_hbm.at[c, pl.ds(s * RPT, RPT)])


def _prep_body(x_ref, win_ref, w1_ref, parts_ref, t1_ref, dis_ref):
    h0 = jnp.dot(x_ref[...], win_ref[...], preferred_element_type=jnp.float32)
    hw1 = jnp.dot(h0, w1_ref[...], preferred_element_type=jnp.float32)
    deg = jnp.sum(parts_ref[...], axis=0) + 1.0
    dis = lax.rsqrt(deg)[:, None]
    dis_ref[...] = dis
    t1_ref[...] = dis * hw1


def _mid_body(p_ref, t_ref, dis_ref, b_ref, w_ref, tn_ref):
    dis = dis_ref[...]
    h = dis * (p_ref[0] + p_ref[1] + t_ref[...]) + b_ref[...]
    h = jnp.maximum(h, 0.0)
    tn_ref[...] = dis * jnp.dot(h, w_ref[...],
                                preferred_element_type=jnp.float32)


def _final_body(q_ref, t_ref, dis_ref, b_ref, wm1_ref, bm1_ref, gamma_ref,
                beta_ref, wm2_ref, bm2_ref, out_ref):
    dis = dis_ref[...]
    h = dis * (q_ref[0] + q_ref[1] + t_ref[...]) + b_ref[...]
    h = jnp.maximum(h, 0.0)[:N]
    m = jnp.dot(h, wm1_ref[...], preferred_element_type=jnp.float32)
    m = m + bm1_ref[...]
    mu = jnp.mean(m, axis=0)
    var = jnp.mean((m - mu) ** 2, axis=0)
    m = gamma_ref[...] * (m - mu) * lax.rsqrt(var + 1e-5) + beta_ref[...]
    m = jnp.maximum(m, 0.0)
    logits = jnp.dot(m, wm2_ref[...], preferred_element_type=jnp.float32)
    logits = logits + bm2_ref[...]
    logits = logits - jnp.max(logits, axis=-1, keepdims=True)
    ex = jnp.exp(logits)
    out_ref[...] = ex / jnp.sum(ex, axis=-1, keepdims=True)


_vmem = pl.BlockSpec(memory_space=pltpu.VMEM)


def _tc_call(body, num_in, out_shape):
    return pl.pallas_call(
        body,
        in_specs=[_vmem] * num_in,
        out_specs=_vmem if not isinstance(out_shape, (tuple, list)) else
        [_vmem] * len(out_shape),
        out_shape=out_shape,
    )


def kernel(x, edge_index, W_in, W1, b1, W2, b2, Wm1, bm1, gamma, beta,
           Wm2, bm2):
    src = edge_index[0].astype(jnp.int32)
    dst = edge_index[1].astype(jnp.int32)
    pad = jnp.full((EP - E,), N, dtype=jnp.int32)
    src_p = jnp.concatenate([src, pad])
    dst_p = jnp.concatenate([dst, pad])
    wid_slabs = []
    off = 0
    for _s in range(NS):
        for _c in range(NC):
            cpt = CPT0 if _c == 0 else CPT1
            seg = jnp.stack(
                [src_p[off:off + cpt * CHUNK].reshape(-1, G, CHUNK),
                 dst_p[off:off + cpt * CHUNK].reshape(-1, G, CHUNK)], axis=2)
            seg = jnp.pad(seg, ((0, NGRP_MAX - cpt // G), (0, 0), (0, 0),
                                (0, 0)), constant_values=N)
            wid_slabs.append(seg)
            off += cpt * CHUNK
    idx5 = jnp.stack(wid_slabs)
    x_p = jnp.pad(x, ((0, N_PAD - N), (0, 0)))
    zrows = jnp.zeros((RPT, D), jnp.float32)

    parts = _deg_kernel(dst)

    t1, dis = _tc_call(
        _prep_body, 4,
        [jax.ShapeDtypeStruct((N_PAD, D), jnp.float32),
         jax.ShapeDtypeStruct((N_PAD, 1), jnp.float32)],
    )(x_p, W_in, W1, parts)

    p1 = _prop_kernel(t1, idx5, zrows)

    t2 = _tc_call(
        _mid_body, 5, jax.ShapeDtypeStruct((N_PAD, D), jnp.float32),
    )(p1, t1, dis, b1, W2)

    p2 = _prop_kernel(t2, idx5, zrows)

    out = _tc_call(
        _final_body, 10, jax.ShapeDtypeStruct((N, NCT), jnp.float32),
    )(p2, t2, dis, b2, Wm1, bm1, gamma, beta, Wm2, bm2)
    return out
